# baseline (speedup 1.0000x reference)
import dataclasses

import jax
import jax.numpy as jnp
from jax import lax
from jax.experimental import pallas as pl
from jax.experimental.pallas import tpu as pltpu
from jax.experimental.pallas import tpu_sc as plsc

MAX_CONTEXT = 200
PADDING_TOKEN = 0

_NC = 2
_NS = 16
_NW = _NC * _NS
_GRP = 16

_CHUNK_OFFS = tuple(range(0, 192, 16)) + (184,)


def _sc_kernel_call(inputs, emb_flat, batch, seq, row_elems):
    rpw = batch // _NW
    mesh = plsc.VectorSubcoreMesh(core_axis_name="c", subcore_axis_name="s")

    def body(inp_hbm, emb_hbm, out_hbm, emb_v, inp_v, dirty_v):
        c = lax.axis_index("c")
        s = lax.axis_index("s")
        wid = s * _NC + c
        base = wid * rpw
        pltpu.sync_copy(emb_hbm, emb_v)

        @pl.loop(0, rpw // _GRP)
        def _group(g):
            row0 = base + g * _GRP
            pltpu.sync_copy(inp_hbm.at[pl.ds(row0, _GRP)], inp_v)
            for r in range(_GRP):
                ok = None
                for off in _CHUNK_OFFS:
                    nz = inp_v[r, pl.ds(off, 16)] != PADDING_TOKEN
                    ok = nz if ok is None else jnp.logical_and(ok, nz)
                clean = jnp.all(ok)

                @pl.when(clean)
                def _fast():
                    pltpu.sync_copy(emb_v, out_hbm.at[pl.ds(row0 + r, 1)])

                @pl.when(jnp.logical_not(clean))
                def _masked():
                    lane = jnp.arange(16, dtype=jnp.int32)
                    zero16 = jnp.zeros((16,), jnp.int32)

                    @pl.loop(0, row_elems // 16)
                    def _chunk(ch):
                        col = ch * 16
                        tok = col // 64
                        tok16 = jnp.broadcast_to(tok, (16,))
                        r16 = jnp.broadcast_to(jnp.int32(r), (16,))
                        tokval = plsc.load_gather(inp_v, [r16, tok16])
                        m = (tokval != PADDING_TOKEN).astype(jnp.float32)
                        cols = col + lane
                        ev = plsc.load_gather(emb_v, [zero16, cols])
                        plsc.store_scatter(dirty_v, [zero16, cols], ev * m)

                    pltpu.sync_copy(dirty_v, out_hbm.at[pl.ds(row0 + r, 1)])

    cp = pltpu.CompilerParams()
    if "needs_layout_passes" in pltpu.CompilerParams.__dataclass_fields__:
        cp = dataclasses.replace(cp, needs_layout_passes=False)
    kern = pl.kernel(
        body,
        out_type=jax.ShapeDtypeStruct((batch, row_elems), jnp.float32),
        mesh=mesh,
        compiler_params=cp,
        scratch_types=[
            pltpu.VMEM((1, row_elems), jnp.float32),
            pltpu.VMEM((_GRP, seq), jnp.int32),
            pltpu.VMEM((1, row_elems), jnp.float32),
        ],
    )
    return kern(inputs, emb_flat)


def kernel(inputs, embedding_matrix):
    if inputs.shape[1] > MAX_CONTEXT:
        inputs = inputs[:, -MAX_CONTEXT:]
    batch, seq = inputs.shape
    dim = embedding_matrix.shape[1]
    row_elems = seq * dim
    emb_flat = embedding_matrix.reshape(1, row_elems)
    out2 = _sc_kernel_call(inputs, emb_flat, batch, seq, row_elems)
    return out2.reshape(batch, seq, dim)


# speedup vs baseline: 1.0469x; 1.0469x over previous
import dataclasses

import jax
import jax.numpy as jnp
from jax import lax
from jax.experimental import pallas as pl
from jax.experimental.pallas import tpu as pltpu
from jax.experimental.pallas import tpu_sc as plsc

MAX_CONTEXT = 200
PADDING_TOKEN = 0

_NC = 2
_NS = 16
_NW = _NC * _NS
_GRP = 16

_CHUNK_OFFS = tuple(range(0, 192, 16)) + (184,)


def _sc_kernel_call(inputs, emb_flat, batch, seq, row_elems):
    rpw = batch // _NW
    mesh = plsc.VectorSubcoreMesh(core_axis_name="c", subcore_axis_name="s")

    def body(inp_hbm, emb_hbm, out_hbm, emb_v, inp_v, dirty_v, sem):
        c = lax.axis_index("c")
        s = lax.axis_index("s")
        wid = s * _NC + c
        base = wid * rpw
        pltpu.sync_copy(emb_hbm, emb_v)

        def _drain(count):
            def _w(_, x):
                pltpu.make_async_copy(emb_v, out_hbm.at[pl.ds(base, 1)], sem).wait()
                return x

            lax.fori_loop(0, count, _w, 0)

        def _group(g, prev_fired):
            _drain(prev_fired)
            row0 = base + g * _GRP
            pltpu.sync_copy(inp_hbm.at[pl.ds(row0, _GRP)], inp_v)
            fired = jnp.int32(0)
            for r in range(_GRP):
                ok = None
                for off in _CHUNK_OFFS:
                    nz = inp_v[r, pl.ds(off, 16)] != PADDING_TOKEN
                    ok = nz if ok is None else jnp.logical_and(ok, nz)
                clean = jnp.all(ok)
                fired = fired + clean.astype(jnp.int32)

                @pl.when(clean)
                def _fast():
                    pltpu.make_async_copy(
                        emb_v, out_hbm.at[pl.ds(row0 + r, 1)], sem
                    ).start()

                @pl.when(jnp.logical_not(clean))
                def _masked():
                    lane = jnp.arange(16, dtype=jnp.int32)
                    zero16 = jnp.zeros((16,), jnp.int32)

                    @pl.loop(0, row_elems // 16)
                    def _chunk(ch):
                        col = ch * 16
                        tok = col // 64
                        tok16 = jnp.broadcast_to(tok, (16,))
                        r16 = jnp.broadcast_to(jnp.int32(r), (16,))
                        tokval = plsc.load_gather(inp_v, [r16, tok16])
                        m = (tokval != PADDING_TOKEN).astype(jnp.float32)
                        cols = col + lane
                        ev = plsc.load_gather(emb_v, [zero16, cols])
                        plsc.store_scatter(dirty_v, [zero16, cols], ev * m)

                    pltpu.sync_copy(dirty_v, out_hbm.at[pl.ds(row0 + r, 1)])

            return fired

        last = lax.fori_loop(0, rpw // _GRP, _group, jnp.int32(0))
        _drain(last)

    cp = pltpu.CompilerParams()
    if "needs_layout_passes" in pltpu.CompilerParams.__dataclass_fields__:
        cp = dataclasses.replace(cp, needs_layout_passes=False)
    kern = pl.kernel(
        body,
        out_type=jax.ShapeDtypeStruct((batch, row_elems), jnp.float32),
        mesh=mesh,
        compiler_params=cp,
        scratch_types=[
            pltpu.VMEM((1, row_elems), jnp.float32),
            pltpu.VMEM((_GRP, seq), jnp.int32),
            pltpu.VMEM((1, row_elems), jnp.float32),
            pltpu.SemaphoreType.DMA,
        ],
    )
    return kern(inputs, emb_flat)


def kernel(inputs, embedding_matrix):
    if inputs.shape[1] > MAX_CONTEXT:
        inputs = inputs[:, -MAX_CONTEXT:]
    batch, seq = inputs.shape
    dim = embedding_matrix.shape[1]
    row_elems = seq * dim
    emb_flat = embedding_matrix.reshape(1, row_elems)
    out2 = _sc_kernel_call(inputs, emb_flat, batch, seq, row_elems)
    return out2.reshape(batch, seq, dim)
